# Initial kernel scaffold; baseline (speedup 1.0000x reference)
import jax
import jax.numpy as jnp
from jax.experimental import pallas as pl


def kernel(x, edge_index, batch, W1, b1, W2, b2, W3, b3, Wc, bc):
    raise NotImplementedError("write your pallas kernel here")


# speedup vs baseline: 18.0517x; 18.0517x over previous
import functools

import jax
import jax.numpy as jnp
from jax import lax
from jax.experimental import pallas as pl
from jax.experimental.pallas import tpu as pltpu
from jax.experimental.pallas import tpu_sc as plsc

N = 50000
E = 800000
IN_DIM = 128
HID = 64
HALF = 32
NG = 64
NC = 8

NP = 50048
EP = 802816
EROWS = EP // 128
ACC_N = 50048
TRASH = 50000

f32 = jnp.float32


def _sc_deg_body(colp_hbm, out_hbm, acc, colbuf, ones_v, zero_v, stage_v, sem):
    c = lax.axis_index("c")
    s = lax.axis_index("s")
    wid = s * 2 + c

    @pl.loop(0, 8)
    def _fill(j):
        ones_v[pl.ds(j * 16, 16)] = jnp.ones((16,), f32)
        zero_v[pl.ds(j * 16, 16)] = jnp.zeros((16,), f32)

    @pl.loop(0, 25)
    def _zero(k):
        ck = s + 16 * k

        @pl.when(ck < 391)
        def _():
            pltpu.sync_copy(zero_v, acc.at[pl.ds(ck * 128, 128)])

    plsc.subcore_barrier()

    @pl.loop(0, 49)
    def _outer(i):
        rb = wid * 196 + i * 4
        pltpu.sync_copy(colp_hbm.at[pl.ds(rb, 4)], colbuf)
        for r in range(4):
            pltpu.sync_copy(ones_v, acc.at[colbuf.at[r]], add=True)

    plsc.subcore_barrier()

    pltpu.sync_copy(acc.at[pl.ds(s * 3128, 3128)], stage_v)
    pltpu.sync_copy(stage_v, out_hbm.at[pl.ds(c * NP + s * 3128, 3128)])


def _sc_agg_body(rowp_hbm, colp_hbm, hp_hbm, out_hbm, acc,
                 rowbuf, colbuf, rowadj, rows_v, zero_v, sem):
    c = lax.axis_index("c")
    s = lax.axis_index("s")
    off = c * NP

    @pl.loop(0, 128)
    def _fillz(r):
        zero_v[r, pl.ds(0, 16)] = jnp.zeros((16,), f32)
        zero_v[r, pl.ds(16, 16)] = jnp.zeros((16,), f32)

    @pl.loop(0, 25)
    def _zero(k):
        ck = s + 16 * k

        @pl.when(ck < 391)
        def _():
            pltpu.sync_copy(zero_v, acc.at[pl.ds(ck * 128, 128)])

    plsc.subcore_barrier()

    @pl.loop(0, 98)
    def _outer(i):
        rb = s * 392 + i * 4
        pltpu.sync_copy(rowp_hbm.at[pl.ds(rb, 4)], rowbuf)
        pltpu.sync_copy(colp_hbm.at[pl.ds(rb, 4)], colbuf)
        for r in range(4):
            for j in range(8):
                rowadj[r, pl.ds(j * 16, 16)] = rowbuf[r, pl.ds(j * 16, 16)] + off
        descs = []
        for r in range(4):
            descs.append(pltpu.async_copy(
                hp_hbm.at[rowadj.at[r]], rows_v.at[pl.ds(r * 128, 128)], sem))
        for d in descs:
            d.wait()
        for r in range(4):
            pltpu.sync_copy(rows_v.at[pl.ds(r * 128, 128)],
                            acc.at[colbuf.at[r]], add=True)

    plsc.subcore_barrier()

    @pl.loop(0, 17)
    def _out(k):
        base = s * 3128 + k * 184
        pltpu.sync_copy(acc.at[pl.ds(base, 184)],
                        rows_v.at[pl.ds(0, 184)])
        pltpu.sync_copy(rows_v.at[pl.ds(0, 184)],
                        out_hbm.at[pl.ds(c * NP + base, 184)])


_SC_MESH = plsc.VectorSubcoreMesh(core_axis_name="c", subcore_axis_name="s")


def _build_sc_deg():
    return pl.kernel(
        _sc_deg_body,
        out_type=jax.ShapeDtypeStruct((2 * NP,), f32),
        mesh=_SC_MESH,
        scratch_types=[
            pltpu.VMEM_SHARED((ACC_N,), f32),
            pltpu.VMEM((4, 128), jnp.int32),
            pltpu.VMEM((128,), f32),
            pltpu.VMEM((128,), f32),
            pltpu.VMEM((3128,), f32),
            pltpu.SemaphoreType.DMA,
        ],
    )


def _build_sc_agg():
    return pl.kernel(
        _sc_agg_body,
        out_type=jax.ShapeDtypeStruct((2 * NP, HALF), f32),
        mesh=_SC_MESH,
        scratch_types=[
            pltpu.VMEM_SHARED((ACC_N, HALF), f32),
            pltpu.VMEM((4, 128), jnp.int32),
            pltpu.VMEM((4, 128), jnp.int32),
            pltpu.VMEM((4, 128), jnp.int32),
            pltpu.VMEM((512, HALF), f32),
            pltpu.VMEM((128, HALF), f32),
            pltpu.SemaphoreType.DMA,
        ],
        compiler_params=pltpu.CompilerParams(use_tc_tiling_on_sc=False),
    )


_BLK = 1000
_NBLK = N // _BLK


def _tc1_body(x_ref, w_ref, degp_ref, hp_ref, dinv_ref):
    h = jnp.dot(x_ref[...], w_ref[...], preferred_element_type=f32)
    dtot = degp_ref[0] + degp_ref[1] + 1.0
    dinv = lax.rsqrt(dtot)
    dinv_ref[...] = dinv
    hp = h * dinv
    hp_ref[0] = hp[:, :HALF]
    hp_ref[1] = hp[:, HALF:]


def _tc_mid_body(agg_ref, hpp_ref, dinv_ref, b_ref, w_ref, hp_ref):
    a = jnp.concatenate([agg_ref[0], agg_ref[1]], axis=1)
    hpv = jnp.concatenate([hpp_ref[0], hpp_ref[1]], axis=1)
    s = jax.nn.relu(dinv_ref[...] * (a + hpv) + b_ref[...])
    h = jnp.dot(s, w_ref[...], preferred_element_type=f32)
    hp = h * dinv_ref[...]
    hp_ref[0] = hp[:, :HALF]
    hp_ref[1] = hp[:, HALF:]


def _tc_final_body(agg_ref, hpp_ref, dinv_ref, b_ref, batch_ref, wc_ref,
                   bc_ref, out_ref, psum, cnt):
    i = pl.program_id(0)
    a = jnp.concatenate([agg_ref[0], agg_ref[1]], axis=1)
    hpv = jnp.concatenate([hpp_ref[0], hpp_ref[1]], axis=1)
    h = jax.nn.relu(dinv_ref[...] * (a + hpv) + b_ref[...])
    onehot_t = (lax.broadcasted_iota(jnp.int32, (NG, _BLK), 0)
                == batch_ref[0]).astype(f32)
    ps = jnp.dot(onehot_t, h, preferred_element_type=f32)
    ct = jnp.sum(onehot_t, axis=1, keepdims=True)

    @pl.when(i == 0)
    def _init():
        psum[...] = ps
        cnt[...] = ct

    @pl.when(i > 0)
    def _acc():
        psum[...] += ps
        cnt[...] += ct

    @pl.when(i == _NBLK - 1)
    def _fin():
        pooled = psum[...] / jnp.maximum(cnt[...], 1.0)
        out_ref[...] = (jnp.dot(pooled, wc_ref[...],
                                preferred_element_type=f32) + bc_ref[...])


def _make_tc1():
    return pl.pallas_call(
        _tc1_body,
        grid=(_NBLK,),
        in_specs=[
            pl.BlockSpec((_BLK, IN_DIM), lambda i: (i, 0)),
            pl.BlockSpec((IN_DIM, HID), lambda i: (0, 0)),
            pl.BlockSpec((2, _BLK, 1), lambda i: (0, i, 0)),
        ],
        out_specs=[
            pl.BlockSpec((2, _BLK, HALF), lambda i: (0, i, 0)),
            pl.BlockSpec((_BLK, 1), lambda i: (i, 0)),
        ],
        out_shape=[
            jax.ShapeDtypeStruct((2, NP, HALF), f32),
            jax.ShapeDtypeStruct((N, 1), f32),
        ],
    )


def _make_tc_mid():
    return pl.pallas_call(
        _tc_mid_body,
        grid=(_NBLK,),
        in_specs=[
            pl.BlockSpec((2, _BLK, HALF), lambda i: (0, i, 0)),
            pl.BlockSpec((2, _BLK, HALF), lambda i: (0, i, 0)),
            pl.BlockSpec((_BLK, 1), lambda i: (i, 0)),
            pl.BlockSpec((1, HID), lambda i: (0, 0)),
            pl.BlockSpec((HID, HID), lambda i: (0, 0)),
        ],
        out_specs=[
            pl.BlockSpec((2, _BLK, HALF), lambda i: (0, i, 0)),
        ],
        out_shape=[jax.ShapeDtypeStruct((2, NP, HALF), f32)],
    )


def _make_tc_final():
    return pl.pallas_call(
        _tc_final_body,
        grid=(_NBLK,),
        in_specs=[
            pl.BlockSpec((2, _BLK, HALF), lambda i: (0, i, 0)),
            pl.BlockSpec((2, _BLK, HALF), lambda i: (0, i, 0)),
            pl.BlockSpec((_BLK, 1), lambda i: (i, 0)),
            pl.BlockSpec((1, HID), lambda i: (0, 0)),
            pl.BlockSpec((1, 1, _BLK), lambda i: (i, 0, 0)),
            pl.BlockSpec((HID, NC), lambda i: (0, 0)),
            pl.BlockSpec((1, NC), lambda i: (0, 0)),
        ],
        out_specs=pl.BlockSpec((NG, NC), lambda i: (0, 0)),
        out_shape=jax.ShapeDtypeStruct((NG, NC), f32),
        scratch_shapes=[
            pltpu.VMEM((NG, HID), f32),
            pltpu.VMEM((NG, 1), f32),
        ],
    )


@jax.jit
def kernel(x, edge_index, batch, W1, b1, W2, b2, W3, b3, Wc, bc):
    sc_deg = _build_sc_deg()
    sc_agg = _build_sc_agg()
    tc1 = _make_tc1()
    tc_mid = _make_tc_mid()
    tc_final = _make_tc_final()

    pad = EP - E
    rowp = jnp.concatenate(
        [edge_index[0], jnp.zeros((pad,), jnp.int32)]).reshape(EROWS, 128)
    colp = jnp.concatenate(
        [edge_index[1], jnp.full((pad,), TRASH, jnp.int32)]).reshape(EROWS, 128)
    batch3 = batch.reshape(_NBLK, 1, _BLK)

    degp = sc_deg(colp).reshape(2, NP, 1)
    hp1, dinv = tc1(x, W1, degp)
    agg1 = sc_agg(rowp, colp, hp1.reshape(2 * NP, HALF)).reshape(2, NP, HALF)
    hp2, = tc_mid(agg1, hp1, dinv, b1.reshape(1, HID), W2)
    agg2 = sc_agg(rowp, colp, hp2.reshape(2 * NP, HALF)).reshape(2, NP, HALF)
    hp3, = tc_mid(agg2, hp2, dinv, b2.reshape(1, HID), W3)
    agg3 = sc_agg(rowp, colp, hp3.reshape(2 * NP, HALF)).reshape(2, NP, HALF)
    return tc_final(agg3, hp3, dinv, b3.reshape(1, HID), batch3,
                    Wc, bc.reshape(1, NC))


# speedup vs baseline: 23.0018x; 1.2742x over previous
import functools

import jax
import jax.numpy as jnp
from jax import lax
from jax.experimental import pallas as pl
from jax.experimental.pallas import tpu as pltpu
from jax.experimental.pallas import tpu_sc as plsc

N = 50000
E = 800000
IN_DIM = 128
HID = 64
HALF = 32
NG = 64
NC = 8

NP = 50048
EP = 802816
EROWS = EP // 128
ACC_N = 50048
TRASH = 50000

f32 = jnp.float32


def _sc_deg_body(colp_hbm, out_hbm, acc, colbuf, ones_v, zero_v, stage_v, sem):
    c = lax.axis_index("c")
    s = lax.axis_index("s")
    wid = s * 2 + c

    @pl.loop(0, 8)
    def _fill(j):
        ones_v[pl.ds(j * 16, 16)] = jnp.ones((16,), f32)
        zero_v[pl.ds(j * 16, 16)] = jnp.zeros((16,), f32)

    @pl.loop(0, 25)
    def _zero(k):
        ck = s + 16 * k

        @pl.when(ck < 391)
        def _():
            pltpu.sync_copy(zero_v, acc.at[pl.ds(ck * 128, 128)])

    plsc.subcore_barrier()

    @pl.loop(0, 49)
    def _outer(i):
        rb = wid * 196 + i * 4
        pltpu.sync_copy(colp_hbm.at[pl.ds(rb, 4)], colbuf)
        for r in range(4):
            pltpu.sync_copy(ones_v, acc.at[colbuf.at[r]], add=True)

    plsc.subcore_barrier()

    pltpu.sync_copy(acc.at[pl.ds(s * 3128, 3128)], stage_v)
    pltpu.sync_copy(stage_v, out_hbm.at[pl.ds(c * NP + s * 3128, 3128)])


def _sc_agg_body(rowp_hbm, colp_hbm, hp_hbm, out_hbm, acc,
                 rowA, colA, rowB, colB, rowsA, rowsB, zero_v,
                 siA, siB, sgA, sgB, ssA, ssB):
    c = lax.axis_index("c")
    s = lax.axis_index("s")

    @pl.loop(0, 64)
    def _fillz(r):
        zero_v[r, pl.ds(0, 16)] = jnp.zeros((16,), f32)
        zero_v[r, pl.ds(16, 16)] = jnp.zeros((16,), f32)

    @pl.loop(0, 49)
    def _zero(k):
        ck = s + 16 * k

        @pl.when(ck < 782)
        def _():
            pltpu.sync_copy(zero_v, acc.at[pl.ds(ck * 64, 64)])

    plsc.subcore_barrier()

    def fire_idx(g, rowb, colb, sem):
        rb = s * 392 + g * 2
        pltpu.async_copy(rowp_hbm.at[pl.ds(rb, 2)], rowb, sem)
        pltpu.async_copy(colp_hbm.at[pl.ds(rb, 2)], colb, sem)

    def wait_idx(rowb, colb, sem):
        pltpu.make_async_copy(rowp_hbm.at[pl.ds(0, 2)], rowb, sem).wait()
        pltpu.make_async_copy(colp_hbm.at[pl.ds(0, 2)], colb, sem).wait()

    def fire_g(rowb, rowsb, sem):
        for r in range(2):
            pltpu.async_copy(hp_hbm.at[c].at[rowb.at[r]],
                             rowsb.at[pl.ds(r * 128, 128)], sem)

    def wait_g(rowb, rowsb, sem):
        for r in range(2):
            pltpu.make_async_copy(hp_hbm.at[c].at[rowb.at[r]],
                                  rowsb.at[pl.ds(r * 128, 128)], sem).wait()

    def fire_s(colb, rowsb, sem):
        for r in range(2):
            pltpu.async_copy(rowsb.at[pl.ds(r * 128, 128)],
                             acc.at[colb.at[r]], sem, add=True)

    def wait_s(colb, rowsb, sem):
        for r in range(2):
            pltpu.make_async_copy(rowsb.at[pl.ds(r * 128, 128)],
                                  acc.at[colb.at[r]], sem).wait()

    fire_idx(0, rowA, colA, siA)

    @pl.loop(0, 98)
    def _outer(i):
        @pl.when(i > 0)
        def _wsa():
            wait_s(colA, rowsA, ssA)

        @pl.when(i > 0)
        def _fia():
            fire_idx(2 * i, rowA, colA, siA)

        @pl.when(i > 0)
        def _sb():
            wait_g(rowB, rowsB, sgB)
            fire_s(colB, rowsB, ssB)

        wait_idx(rowA, colA, siA)
        fire_g(rowA, rowsA, sgA)

        @pl.when(i > 0)
        def _wsb():
            wait_s(colB, rowsB, ssB)

        fire_idx(2 * i + 1, rowB, colB, siB)
        wait_idx(rowB, colB, siB)
        fire_g(rowB, rowsB, sgB)

        wait_g(rowA, rowsA, sgA)
        fire_s(colA, rowsA, ssA)

    wait_g(rowB, rowsB, sgB)
    fire_s(colB, rowsB, ssB)
    wait_s(colA, rowsA, ssA)
    wait_s(colB, rowsB, ssB)

    plsc.subcore_barrier()

    @pl.loop(0, 17)
    def _out(k):
        base = s * 3128 + k * 184
        pltpu.sync_copy(acc.at[pl.ds(base, 184)],
                        rowsA.at[pl.ds(0, 184)])
        pltpu.sync_copy(rowsA.at[pl.ds(0, 184)],
                        out_hbm.at[c, pl.ds(base, 184)])


_SC_MESH = plsc.VectorSubcoreMesh(core_axis_name="c", subcore_axis_name="s")


def _build_sc_deg():
    return pl.kernel(
        _sc_deg_body,
        out_type=jax.ShapeDtypeStruct((2 * NP,), f32),
        mesh=_SC_MESH,
        scratch_types=[
            pltpu.VMEM_SHARED((ACC_N,), f32),
            pltpu.VMEM((4, 128), jnp.int32),
            pltpu.VMEM((128,), f32),
            pltpu.VMEM((128,), f32),
            pltpu.VMEM((3128,), f32),
            pltpu.SemaphoreType.DMA,
        ],
    )


def _build_sc_agg():
    return pl.kernel(
        _sc_agg_body,
        out_type=jax.ShapeDtypeStruct((2, NP, HALF), f32),
        mesh=_SC_MESH,
        scratch_types=[
            pltpu.VMEM_SHARED((ACC_N, HALF), f32),
            pltpu.VMEM((2, 128), jnp.int32),
            pltpu.VMEM((2, 128), jnp.int32),
            pltpu.VMEM((2, 128), jnp.int32),
            pltpu.VMEM((2, 128), jnp.int32),
            pltpu.VMEM((256, HALF), f32),
            pltpu.VMEM((256, HALF), f32),
            pltpu.VMEM((64, HALF), f32),
            pltpu.SemaphoreType.DMA,
            pltpu.SemaphoreType.DMA,
            pltpu.SemaphoreType.DMA,
            pltpu.SemaphoreType.DMA,
            pltpu.SemaphoreType.DMA,
            pltpu.SemaphoreType.DMA,
        ],
        compiler_params=pltpu.CompilerParams(use_tc_tiling_on_sc=False),
    )


_BLK = 1000
_NBLK = N // _BLK


def _tc1_body(x_ref, w_ref, degp_ref, hp_ref, dinv_ref):
    h = jnp.dot(x_ref[...], w_ref[...], preferred_element_type=f32)
    dtot = degp_ref[0] + degp_ref[1] + 1.0
    dinv = lax.rsqrt(dtot)
    dinv_ref[...] = dinv
    hp = h * dinv
    hp_ref[0] = hp[:, :HALF]
    hp_ref[1] = hp[:, HALF:]


def _tc_mid_body(agg_ref, hpp_ref, dinv_ref, b_ref, w_ref, hp_ref):
    a = jnp.concatenate([agg_ref[0], agg_ref[1]], axis=1)
    hpv = jnp.concatenate([hpp_ref[0], hpp_ref[1]], axis=1)
    s = jax.nn.relu(dinv_ref[...] * (a + hpv) + b_ref[...])
    h = jnp.dot(s, w_ref[...], preferred_element_type=f32)
    hp = h * dinv_ref[...]
    hp_ref[0] = hp[:, :HALF]
    hp_ref[1] = hp[:, HALF:]


def _tc_final_body(agg_ref, hpp_ref, dinv_ref, b_ref, batch_ref, wc_ref,
                   bc_ref, out_ref, psum, cnt):
    i = pl.program_id(0)
    a = jnp.concatenate([agg_ref[0], agg_ref[1]], axis=1)
    hpv = jnp.concatenate([hpp_ref[0], hpp_ref[1]], axis=1)
    h = jax.nn.relu(dinv_ref[...] * (a + hpv) + b_ref[...])
    onehot_t = (lax.broadcasted_iota(jnp.int32, (NG, _BLK), 0)
                == batch_ref[0]).astype(f32)
    ps = jnp.dot(onehot_t, h, preferred_element_type=f32)
    ct = jnp.sum(onehot_t, axis=1, keepdims=True)

    @pl.when(i == 0)
    def _init():
        psum[...] = ps
        cnt[...] = ct

    @pl.when(i > 0)
    def _acc():
        psum[...] += ps
        cnt[...] += ct

    @pl.when(i == _NBLK - 1)
    def _fin():
        pooled = psum[...] / jnp.maximum(cnt[...], 1.0)
        out_ref[...] = (jnp.dot(pooled, wc_ref[...],
                                preferred_element_type=f32) + bc_ref[...])


def _make_tc1():
    return pl.pallas_call(
        _tc1_body,
        grid=(_NBLK,),
        in_specs=[
            pl.BlockSpec((_BLK, IN_DIM), lambda i: (i, 0)),
            pl.BlockSpec((IN_DIM, HID), lambda i: (0, 0)),
            pl.BlockSpec((2, _BLK, 1), lambda i: (0, i, 0)),
        ],
        out_specs=[
            pl.BlockSpec((2, _BLK, HALF), lambda i: (0, i, 0)),
            pl.BlockSpec((_BLK, 1), lambda i: (i, 0)),
        ],
        out_shape=[
            jax.ShapeDtypeStruct((2, NP, HALF), f32),
            jax.ShapeDtypeStruct((N, 1), f32),
        ],
    )


def _make_tc_mid():
    return pl.pallas_call(
        _tc_mid_body,
        grid=(_NBLK,),
        in_specs=[
            pl.BlockSpec((2, _BLK, HALF), lambda i: (0, i, 0)),
            pl.BlockSpec((2, _BLK, HALF), lambda i: (0, i, 0)),
            pl.BlockSpec((_BLK, 1), lambda i: (i, 0)),
            pl.BlockSpec((1, HID), lambda i: (0, 0)),
            pl.BlockSpec((HID, HID), lambda i: (0, 0)),
        ],
        out_specs=[
            pl.BlockSpec((2, _BLK, HALF), lambda i: (0, i, 0)),
        ],
        out_shape=[jax.ShapeDtypeStruct((2, NP, HALF), f32)],
    )


def _make_tc_final():
    return pl.pallas_call(
        _tc_final_body,
        grid=(_NBLK,),
        in_specs=[
            pl.BlockSpec((2, _BLK, HALF), lambda i: (0, i, 0)),
            pl.BlockSpec((2, _BLK, HALF), lambda i: (0, i, 0)),
            pl.BlockSpec((_BLK, 1), lambda i: (i, 0)),
            pl.BlockSpec((1, HID), lambda i: (0, 0)),
            pl.BlockSpec((1, 1, _BLK), lambda i: (i, 0, 0)),
            pl.BlockSpec((HID, NC), lambda i: (0, 0)),
            pl.BlockSpec((1, NC), lambda i: (0, 0)),
        ],
        out_specs=pl.BlockSpec((NG, NC), lambda i: (0, 0)),
        out_shape=jax.ShapeDtypeStruct((NG, NC), f32),
        scratch_shapes=[
            pltpu.VMEM((NG, HID), f32),
            pltpu.VMEM((NG, 1), f32),
        ],
    )


@jax.jit
def kernel(x, edge_index, batch, W1, b1, W2, b2, W3, b3, Wc, bc):
    sc_deg = _build_sc_deg()
    sc_agg = _build_sc_agg()
    tc1 = _make_tc1()
    tc_mid = _make_tc_mid()
    tc_final = _make_tc_final()

    pad = EP - E
    rowp = jnp.concatenate(
        [edge_index[0], jnp.zeros((pad,), jnp.int32)]).reshape(EROWS, 128)
    colp = jnp.concatenate(
        [edge_index[1], jnp.full((pad,), TRASH, jnp.int32)]).reshape(EROWS, 128)
    batch3 = batch.reshape(_NBLK, 1, _BLK)

    degp = sc_deg(colp).reshape(2, NP, 1)
    hp1, dinv = tc1(x, W1, degp)
    agg1 = sc_agg(rowp, colp, hp1)
    hp2, = tc_mid(agg1, hp1, dinv, b1.reshape(1, HID), W2)
    agg2 = sc_agg(rowp, colp, hp2)
    hp3, = tc_mid(agg2, hp2, dinv, b2.reshape(1, HID), W3)
    agg3 = sc_agg(rowp, colp, hp3)
    return tc_final(agg3, hp3, dinv, b3.reshape(1, HID), batch3,
                    Wc, bc.reshape(1, NC))
